# Initial kernel scaffold; baseline (speedup 1.0000x reference)
import jax
import jax.numpy as jnp
from jax.experimental import pallas as pl


def kernel(x_user, x_item, rates_src, rates_dst, rev_src, rev_dst, l1_rates_Wself, l1_rates_Wneigh, l1_rates_b, l1_rev_Wself, l1_rev_Wneigh, l1_rev_b, l2_rates_Wself, l2_rates_Wneigh, l2_rates_b, l2_rev_Wself, l2_rev_Wneigh, l2_rev_b):
    raise NotImplementedError("write your pallas kernel here")


# speedup vs baseline: 3.9143x; 3.9143x over previous
import functools

import jax
import jax.numpy as jnp
from jax import lax
from jax.experimental import pallas as pl
from jax.experimental.pallas import tpu as pltpu
from jax.experimental.pallas import tpu_sc as plsc

NU, NI, D, E = 30000, 20000, 128, 300000
NUP, NIP = 30080, 20096
NC, NT = 2, 16
CHUNK = 128
CPT = 152
EPAD = NT * CPT * CHUNK
HALF = D // 2


DBUF = 1888
NQ = 4
QW = D // NQ


def _agg_pass(c, s, n_pad, table4, src2d, dst2d, out_hbm, zacc,
              deg_out, deg_core, src_st, dst_st, rows0, rows1, ones_v,
              dbuf, acc_sh, deg_sh, sg0, sg1):
    rows_pt = n_pad // NT
    row_sl = pl.ds(s * rows_pt, rows_pt)
    pltpu.sync_copy(src2d.at[pl.ds(s * CPT, CPT)], src_st)
    pltpu.sync_copy(dst2d.at[pl.ds(s * CPT, CPT)], dst_st)

    for h in range(2):
        q = 2 * h + c
        pltpu.sync_copy(zacc.at[row_sl], acc_sh.at[row_sl])
        if deg_out is not None and h == 0:
            @pl.when(c == deg_core)
            def _():
                def _zb(i, carry):
                    dbuf[pl.ds(i * 16, 16)] = jnp.zeros((16,), jnp.float32)
                    return carry
                lax.fori_loop(0, DBUF // 16, _zb, 0)
                pltpu.sync_copy(dbuf.at[pl.ds(0, rows_pt)], deg_sh.at[row_sl])

        def _trans(i, carry):
            for j in range(CHUNK // 16):
                sl = pl.ds(j * 16, 16)
                v = src_st[i, sl]
                if h == 0:
                    src_st[i, sl] = v * 4 + c
                else:
                    src_st[i, sl] = v + 2
            return carry
        lax.fori_loop(0, CPT, _trans, 0)

        plsc.subcore_barrier()

        def g_start(i, rbuf, sem):
            pltpu.async_copy(table4.at[src_st.at[i]], rbuf, sem)

        def g_wait(i, rbuf, sem):
            pltpu.make_async_copy(table4.at[src_st.at[i]], rbuf, sem).wait()

        def s_add(i, rbuf):
            pltpu.sync_copy(rbuf, acc_sh.at[dst_st.at[i]], add=True)
            if deg_out is not None and h == 0:
                @pl.when(c == deg_core)
                def _():
                    pltpu.sync_copy(ones_v, deg_sh.at[dst_st.at[i]], add=True)

        g_start(0, rows0, sg0)

        def _step(t, carry):
            i0 = t * 2
            g_wait(i0, rows0, sg0)
            g_start(i0 + 1, rows1, sg1)
            s_add(i0, rows0)
            g_wait(i0 + 1, rows1, sg1)
            @pl.when(i0 + 2 < CPT)
            def _():
                g_start(i0 + 2, rows0, sg0)
            s_add(i0 + 1, rows1)
            return carry
        lax.fori_loop(0, CPT // 2, _step, 0)

        plsc.subcore_barrier()
        pltpu.sync_copy(acc_sh.at[row_sl], out_hbm.at[row_sl, q])
        if deg_out is not None and h == 0:
            @pl.when(c == deg_core)
            def _():
                pltpu.sync_copy(deg_sh.at[row_sl], dbuf.at[pl.ds(0, rows_pt)])
                pltpu.sync_copy(dbuf.at[pl.ds(0, rows_pt)], deg_out.at[row_sl])
        plsc.subcore_barrier()


@functools.lru_cache(maxsize=None)
def _make_sc_kernel(with_deg):
    mesh = plsc.VectorSubcoreMesh(core_axis_name="c", subcore_axis_name="s",
                                  num_cores=NC, num_subcores=NT)
    out_type = [jax.ShapeDtypeStruct((NIP, NQ, QW), jnp.float32),
                jax.ShapeDtypeStruct((NUP, NQ, QW), jnp.float32)]
    if with_deg:
        out_type += [jax.ShapeDtypeStruct((NIP,), jnp.float32),
                     jax.ShapeDtypeStruct((NUP,), jnp.float32)]
    scratch = (
        pltpu.VMEM((CPT, CHUNK), jnp.int32),
        pltpu.VMEM((CPT, CHUNK), jnp.int32),
        pltpu.VMEM((CHUNK, QW), jnp.float32),
        pltpu.VMEM((CHUNK, QW), jnp.float32),
        pltpu.VMEM((CHUNK,), jnp.float32),
        pltpu.VMEM((DBUF,), jnp.float32),
        pltpu.VMEM_SHARED((NUP, QW), jnp.float32),
        pltpu.VMEM_SHARED((NUP,), jnp.float32),
        pltpu.SemaphoreType.DMA,
        pltpu.SemaphoreType.DMA,
    )

    def body(tP, sP, dP, tQ, sQ, dQ, zacc, outP, outQ, *rest):
        if with_deg:
            (degI, degU, src_st, dst_st, rows0, rows1, ones_v, dbuf,
             acc_sh, deg_sh, sg0, sg1) = rest
        else:
            (src_st, dst_st, rows0, rows1, ones_v, dbuf,
             acc_sh, deg_sh, sg0, sg1) = rest
            degI = degU = None
        c = lax.axis_index("c")
        s = lax.axis_index("s")
        if with_deg:
            for j in range(CHUNK // 16):
                ones_v[pl.ds(j * 16, 16)] = jnp.full((16,), 1.0, jnp.float32)
        common = (src_st, dst_st, rows0, rows1, ones_v, dbuf, acc_sh, deg_sh,
                  sg0, sg1)
        _agg_pass(c, s, NIP, tP, sP, dP, outP, zacc, degI, 0, *common)
        _agg_pass(c, s, NUP, tQ, sQ, dQ, outQ, zacc, degU, 1, *common)

    return pl.kernel(body, out_type=tuple(out_type), mesh=mesh,
                     scratch_types=scratch,
                     compiler_params=pltpu.CompilerParams(
                         use_tc_tiling_on_sc=False))


BR = 400


def _sage_block(feat_ref, agg_ref, deg_ref, ws_ref, wn_ref, b_ref, o_ref,
                *, normalize):
    r = 1.0 / jnp.maximum(deg_ref[...], 1.0)
    m = agg_ref[...] * r
    y = (jnp.dot(feat_ref[...], ws_ref[...], preferred_element_type=jnp.float32)
         + jnp.dot(m, wn_ref[...], preferred_element_type=jnp.float32)
         + b_ref[...])
    if normalize:
        n = jnp.sqrt(jnp.sum(y * y, axis=1, keepdims=True))
        y = y / jnp.maximum(n, 1e-12)
    o_ref[...] = y


def _sage_tc(feat, agg, deg2, ws, wn, b2, n_rows, off, normalize):
    return pl.pallas_call(
        functools.partial(_sage_block, normalize=normalize),
        grid=(n_rows // BR,),
        in_specs=[
            pl.BlockSpec((BR, D), lambda i, o=off: (i + o, 0)),
            pl.BlockSpec((BR, D), lambda i, o=off: (i + o, 0)),
            pl.BlockSpec((BR, 1), lambda i, o=off: (i + o, 0)),
            pl.BlockSpec((D, D), lambda i: (0, 0)),
            pl.BlockSpec((D, D), lambda i: (0, 0)),
            pl.BlockSpec((1, D), lambda i: (0, 0)),
        ],
        out_specs=pl.BlockSpec((BR, D), lambda i: (i, 0)),
        out_shape=jax.ShapeDtypeStruct((n_rows, D), jnp.float32),
    )(feat, agg, deg2, ws, wn, b2)


def kernel(x_user, x_item, rates_src, rates_dst, rev_src, rev_dst,
           l1_rates_Wself, l1_rates_Wneigh, l1_rates_b,
           l1_rev_Wself, l1_rev_Wneigh, l1_rev_b,
           l2_rates_Wself, l2_rates_Wneigh, l2_rates_b,
           l2_rev_Wself, l2_rev_Wneigh, l2_rev_b):
    ar = jnp.arange(EPAD - E, dtype=jnp.int32)
    rs = jnp.concatenate([rates_src, ar % 64]).reshape(EPAD // CHUNK, CHUNK)
    rd = jnp.concatenate([rates_dst, NI + (ar % (NIP - NI))]).reshape(EPAD // CHUNK, CHUNK)
    vs = jnp.concatenate([rev_src, ar % 64]).reshape(EPAD // CHUNK, CHUNK)
    vd = jnp.concatenate([rev_dst, NU + (ar % (NUP - NU))]).reshape(EPAD // CHUNK, CHUNK)
    zacc = jnp.zeros((NUP, QW), jnp.float32)
    rs, rd, vs, vd, zacc = lax.optimization_barrier((rs, rd, vs, vd, zacc))

    aggP, aggQ, degI, degU = _make_sc_kernel(True)(
        x_user.reshape(NQ * NU, QW), rs, rd,
        x_item.reshape(NQ * NI, QW), vs, vd, zacc)
    degI2 = degI.reshape(NIP, 1)
    degU2 = degU.reshape(NUP, 1)

    item1 = _sage_tc(x_item, aggP.reshape(NIP, D), degI2,
                     l1_rates_Wself, l1_rates_Wneigh,
                     l1_rates_b.reshape(1, D), NI, 0, False)
    user1 = _sage_tc(x_user, aggQ.reshape(NUP, D), degU2,
                     l1_rev_Wself, l1_rev_Wneigh,
                     l1_rev_b.reshape(1, D), NU, 0, False)

    aggS, aggT = _make_sc_kernel(False)(
        user1.reshape(NQ * NU, QW), rs, rd,
        item1.reshape(NQ * NI, QW), vs, vd, zacc)

    item2 = _sage_tc(item1, aggS.reshape(NIP, D), degI2,
                     l2_rates_Wself, l2_rates_Wneigh,
                     l2_rates_b.reshape(1, D), NI, 0, True)
    user2t = _sage_tc(user1, aggT.reshape(NUP, D), degU2,
                      l2_rev_Wself, l2_rev_Wneigh,
                      l2_rev_b.reshape(1, D), NU - NI, NI // BR, True)
    return jnp.concatenate([item2, user2t], axis=0)


# speedup vs baseline: 5.3638x; 1.3703x over previous
import functools

import jax
import jax.numpy as jnp
from jax import lax
from jax.experimental import pallas as pl
from jax.experimental.pallas import tpu as pltpu
from jax.experimental.pallas import tpu_sc as plsc

NU, NI, D, E = 30000, 20000, 128, 300000
NUP, NIP = 30080, 20096
NC, NT = 2, 16
CHUNK = 128
CPT = 152
EPAD = NT * CPT * CHUNK
HALF = D // 2


DBUF = 1888
NQ = 4
QW = D // NQ


NBUF = 4


def _agg_pass(c, s, n_pad, table4, src2d, dst2d, out_hbm, zacc,
              deg_out, deg_core, src_st, dst_st, rows, ones_v,
              dbuf, acc_sh, deg_sh, sg, ss):
    rows_pt = n_pad // NT
    row_sl = pl.ds(s * rows_pt, rows_pt)
    pltpu.sync_copy(src2d.at[pl.ds(s * CPT, CPT)], src_st)
    pltpu.sync_copy(dst2d.at[pl.ds(s * CPT, CPT)], dst_st)

    for h in range(2):
        q = 2 * h + c
        pltpu.sync_copy(zacc.at[row_sl], acc_sh.at[row_sl])
        if deg_out is not None and h == 0:
            @pl.when(c == deg_core)
            def _():
                def _zb(i, carry):
                    dbuf[pl.ds(i * 16, 16)] = jnp.zeros((16,), jnp.float32)
                    return carry
                lax.fori_loop(0, DBUF // 16, _zb, 0)
                pltpu.sync_copy(dbuf.at[pl.ds(0, rows_pt)], deg_sh.at[row_sl])

        def _trans(i, carry):
            for j in range(CHUNK // 16):
                sl = pl.ds(j * 16, 16)
                v = src_st[i, sl]
                if h == 0:
                    src_st[i, sl] = v * 4 + c
                else:
                    src_st[i, sl] = v + 2
            return carry
        lax.fori_loop(0, CPT, _trans, 0)

        plsc.subcore_barrier()

        def g_start(i, b):
            pltpu.async_copy(table4.at[src_st.at[i]], rows[b], sg[b])

        def g_wait(i, b):
            pltpu.make_async_copy(table4.at[src_st.at[i]], rows[b],
                                  sg[b]).wait()

        def s_start(i, b):
            pltpu.async_copy(rows[b], acc_sh.at[dst_st.at[i]], ss[b],
                             add=True)
            if deg_out is not None and h == 0:
                @pl.when(c == deg_core)
                def _():
                    pltpu.sync_copy(ones_v, deg_sh.at[dst_st.at[i]],
                                    add=True)

        def s_wait(i, b):
            pltpu.make_async_copy(rows[b], acc_sh.at[dst_st.at[i]],
                                  ss[b]).wait()

        g_start(0, 0)
        g_start(1, 1)

        def _step(t, carry):
            for k in range(NBUF):
                j = t * NBUF + k
                b = k
                g_wait(j, b)
                s_start(j, b)
                @pl.when(j >= 2)
                def _():
                    s_wait(j - 2, (k + 2) % NBUF)
                @pl.when(j + 2 < CPT)
                def _():
                    g_start(j + 2, (k + 2) % NBUF)
            return carry
        lax.fori_loop(0, CPT // NBUF, _step, 0)
        s_wait(CPT - 2, (CPT - 2) % NBUF)
        s_wait(CPT - 1, (CPT - 1) % NBUF)

        plsc.subcore_barrier()
        pltpu.sync_copy(acc_sh.at[row_sl], out_hbm.at[row_sl, q])
        if deg_out is not None and h == 0:
            @pl.when(c == deg_core)
            def _():
                pltpu.sync_copy(deg_sh.at[row_sl], dbuf.at[pl.ds(0, rows_pt)])
                pltpu.sync_copy(dbuf.at[pl.ds(0, rows_pt)], deg_out.at[row_sl])
        plsc.subcore_barrier()


@functools.lru_cache(maxsize=None)
def _make_sc_kernel(with_deg):
    mesh = plsc.VectorSubcoreMesh(core_axis_name="c", subcore_axis_name="s",
                                  num_cores=NC, num_subcores=NT)
    out_type = [jax.ShapeDtypeStruct((NIP, NQ, QW), jnp.float32),
                jax.ShapeDtypeStruct((NUP, NQ, QW), jnp.float32)]
    if with_deg:
        out_type += [jax.ShapeDtypeStruct((NIP,), jnp.float32),
                     jax.ShapeDtypeStruct((NUP,), jnp.float32)]
    scratch = (
        pltpu.VMEM((CPT, CHUNK), jnp.int32),
        pltpu.VMEM((CPT, CHUNK), jnp.int32),
        *[pltpu.VMEM((CHUNK, QW), jnp.float32) for _ in range(NBUF)],
        pltpu.VMEM((CHUNK,), jnp.float32),
        pltpu.VMEM((DBUF,), jnp.float32),
        pltpu.VMEM_SHARED((NUP, QW), jnp.float32),
        pltpu.VMEM_SHARED((NUP,), jnp.float32),
        *[pltpu.SemaphoreType.DMA for _ in range(2 * NBUF)],
    )

    def body(tP, sP, dP, tQ, sQ, dQ, zacc, outP, outQ, *rest):
        if with_deg:
            degI, degU = rest[0], rest[1]
            rest = rest[2:]
        else:
            degI = degU = None
        src_st, dst_st = rest[0], rest[1]
        rows = list(rest[2:2 + NBUF])
        ones_v, dbuf, acc_sh, deg_sh = rest[2 + NBUF:6 + NBUF]
        sg = list(rest[6 + NBUF:6 + 2 * NBUF])
        ss = list(rest[6 + 2 * NBUF:6 + 3 * NBUF])
        c = lax.axis_index("c")
        s = lax.axis_index("s")
        if with_deg:
            for j in range(CHUNK // 16):
                ones_v[pl.ds(j * 16, 16)] = jnp.full((16,), 1.0, jnp.float32)
        common = (src_st, dst_st, rows, ones_v, dbuf, acc_sh, deg_sh, sg, ss)
        _agg_pass(c, s, NIP, tP, sP, dP, outP, zacc, degI, 0, *common)
        _agg_pass(c, s, NUP, tQ, sQ, dQ, outQ, zacc, degU, 1, *common)

    return pl.kernel(body, out_type=tuple(out_type), mesh=mesh,
                     scratch_types=scratch,
                     compiler_params=pltpu.CompilerParams(
                         use_tc_tiling_on_sc=False))


BR = 400


def _sage_block(feat_ref, agg_ref, deg_ref, ws_ref, wn_ref, b_ref, o_ref,
                *, normalize):
    r = 1.0 / jnp.maximum(deg_ref[...], 1.0)
    m = agg_ref[...] * r
    y = (jnp.dot(feat_ref[...], ws_ref[...], preferred_element_type=jnp.float32)
         + jnp.dot(m, wn_ref[...], preferred_element_type=jnp.float32)
         + b_ref[...])
    if normalize:
        n = jnp.sqrt(jnp.sum(y * y, axis=1, keepdims=True))
        y = y / jnp.maximum(n, 1e-12)
    o_ref[...] = y


def _sage_tc(feat, agg, deg2, ws, wn, b2, n_rows, off, normalize):
    return pl.pallas_call(
        functools.partial(_sage_block, normalize=normalize),
        grid=(n_rows // BR,),
        in_specs=[
            pl.BlockSpec((BR, D), lambda i, o=off: (i + o, 0)),
            pl.BlockSpec((BR, D), lambda i, o=off: (i + o, 0)),
            pl.BlockSpec((BR, 1), lambda i, o=off: (i + o, 0)),
            pl.BlockSpec((D, D), lambda i: (0, 0)),
            pl.BlockSpec((D, D), lambda i: (0, 0)),
            pl.BlockSpec((1, D), lambda i: (0, 0)),
        ],
        out_specs=pl.BlockSpec((BR, D), lambda i: (i, 0)),
        out_shape=jax.ShapeDtypeStruct((n_rows, D), jnp.float32),
    )(feat, agg, deg2, ws, wn, b2)


def kernel(x_user, x_item, rates_src, rates_dst, rev_src, rev_dst,
           l1_rates_Wself, l1_rates_Wneigh, l1_rates_b,
           l1_rev_Wself, l1_rev_Wneigh, l1_rev_b,
           l2_rates_Wself, l2_rates_Wneigh, l2_rates_b,
           l2_rev_Wself, l2_rev_Wneigh, l2_rev_b):
    ar = jnp.arange(EPAD - E, dtype=jnp.int32)
    rs = jnp.concatenate([rates_src, ar % 64]).reshape(EPAD // CHUNK, CHUNK)
    rd = jnp.concatenate([rates_dst, NI + (ar % (NIP - NI))]).reshape(EPAD // CHUNK, CHUNK)
    vs = jnp.concatenate([rev_src, ar % 64]).reshape(EPAD // CHUNK, CHUNK)
    vd = jnp.concatenate([rev_dst, NU + (ar % (NUP - NU))]).reshape(EPAD // CHUNK, CHUNK)
    zacc = jnp.zeros((NUP, QW), jnp.float32)
    rs, rd, vs, vd, zacc = lax.optimization_barrier((rs, rd, vs, vd, zacc))

    aggP, aggQ, degI, degU = _make_sc_kernel(True)(
        x_user.reshape(NQ * NU, QW), rs, rd,
        x_item.reshape(NQ * NI, QW), vs, vd, zacc)
    degI2 = degI.reshape(NIP, 1)
    degU2 = degU.reshape(NUP, 1)

    item1 = _sage_tc(x_item, aggP.reshape(NIP, D), degI2,
                     l1_rates_Wself, l1_rates_Wneigh,
                     l1_rates_b.reshape(1, D), NI, 0, False)
    user1 = _sage_tc(x_user, aggQ.reshape(NUP, D), degU2,
                     l1_rev_Wself, l1_rev_Wneigh,
                     l1_rev_b.reshape(1, D), NU, 0, False)

    aggS, aggT = _make_sc_kernel(False)(
        user1.reshape(NQ * NU, QW), rs, rd,
        item1.reshape(NQ * NI, QW), vs, vd, zacc)

    item2 = _sage_tc(item1, aggS.reshape(NIP, D), degI2,
                     l2_rates_Wself, l2_rates_Wneigh,
                     l2_rates_b.reshape(1, D), NI, 0, True)
    user2t = _sage_tc(user1, aggT.reshape(NUP, D), degU2,
                      l2_rev_Wself, l2_rev_Wneigh,
                      l2_rev_b.reshape(1, D), NU - NI, NI // BR, True)
    return jnp.concatenate([item2, user2t], axis=0)


# speedup vs baseline: 6.5829x; 1.2273x over previous
import functools

import jax
import jax.numpy as jnp
from jax import lax
from jax.experimental import pallas as pl
from jax.experimental.pallas import tpu as pltpu
from jax.experimental.pallas import tpu_sc as plsc

NU, NI, D, E = 30000, 20000, 128, 300000
NUP, NIP = 30080, 20096
NC, NT = 2, 16
CHUNK = 128
CPT = 152
EPAD = NT * CPT * CHUNK
HALF = D // 2


DBUF = 1888
NQ = 4
QW = D // NQ


NBUF = 6
STAG = 3
SD = 3


def _agg_pass(c, s, n_pad, table4, src2d, dst2d, out_hbm, zacc,
              deg_out, deg_core, src_st, dst_st, rows, ones_v,
              dbuf, acc_sh, deg_sh, sg, ss):
    rows_pt = n_pad // NT
    row_sl = pl.ds(s * rows_pt, rows_pt)
    pltpu.sync_copy(src2d.at[pl.ds(s * CPT, CPT)], src_st)
    pltpu.sync_copy(dst2d.at[pl.ds(s * CPT, CPT)], dst_st)

    for h in range(2):
        q = 2 * h + c
        pltpu.sync_copy(zacc.at[row_sl], acc_sh.at[row_sl])
        if deg_out is not None and h == 0:
            @pl.when(c == deg_core)
            def _():
                def _zb(i, carry):
                    dbuf[pl.ds(i * 16, 16)] = jnp.zeros((16,), jnp.float32)
                    return carry
                lax.fori_loop(0, DBUF // 16, _zb, 0)
                pltpu.sync_copy(dbuf.at[pl.ds(0, rows_pt)], deg_sh.at[row_sl])

        def _trans(i, carry):
            for j in range(CHUNK // 16):
                sl = pl.ds(j * 16, 16)
                v = src_st[i, sl]
                if h == 0:
                    src_st[i, sl] = v * 4 + c
                else:
                    src_st[i, sl] = v + 2
            return carry
        lax.fori_loop(0, CPT, _trans, 0)

        plsc.subcore_barrier()

        def _rbuf(b):
            return rows.at[pl.ds(b * CHUNK, CHUNK)]

        def g_start(i, b):
            pltpu.async_copy(table4.at[src_st.at[i]], _rbuf(b), sg[b])

        def g_wait(i, b):
            pltpu.make_async_copy(table4.at[src_st.at[i]], _rbuf(b),
                                  sg[b]).wait()

        def s_start(i, b):
            pltpu.async_copy(_rbuf(b), acc_sh.at[dst_st.at[i]], ss[b],
                             add=True)
            if deg_out is not None and h == 0:
                @pl.when(c == deg_core)
                def _():
                    pltpu.sync_copy(ones_v, deg_sh.at[dst_st.at[i]],
                                    add=True)

        def s_wait(i, b):
            pltpu.make_async_copy(_rbuf(b), acc_sh.at[dst_st.at[i]],
                                  ss[b]).wait()

        def _step(t, carry):
            for k in range(NBUF):
                i = t * NBUF + k
                @pl.when(jnp.logical_and(i >= STAG + SD,
                                         i < CPT + STAG + SD))
                def _(i=i, k=k):
                    s_wait(i - STAG - SD, k)
                @pl.when(i < CPT)
                def _(i=i, k=k):
                    g_start(i, k)
                b1 = (k - STAG) % NBUF
                @pl.when(jnp.logical_and(i >= STAG, i < CPT + STAG))
                def _(i=i, b1=b1):
                    g_wait(i - STAG, b1)
                    s_start(i - STAG, b1)
            return carry
        lax.fori_loop(0, (CPT + STAG + SD + NBUF - 1) // NBUF, _step, 0)

        plsc.subcore_barrier()
        pltpu.sync_copy(acc_sh.at[row_sl], out_hbm.at[row_sl, q])
        if deg_out is not None and h == 0:
            @pl.when(c == deg_core)
            def _():
                pltpu.sync_copy(deg_sh.at[row_sl], dbuf.at[pl.ds(0, rows_pt)])
                pltpu.sync_copy(dbuf.at[pl.ds(0, rows_pt)], deg_out.at[row_sl])
        plsc.subcore_barrier()


@functools.lru_cache(maxsize=None)
def _make_sc_kernel(with_deg):
    mesh = plsc.VectorSubcoreMesh(core_axis_name="c", subcore_axis_name="s",
                                  num_cores=NC, num_subcores=NT)
    out_type = [jax.ShapeDtypeStruct((NIP, NQ, QW), jnp.float32),
                jax.ShapeDtypeStruct((NUP, NQ, QW), jnp.float32)]
    if with_deg:
        out_type += [jax.ShapeDtypeStruct((NIP,), jnp.float32),
                     jax.ShapeDtypeStruct((NUP,), jnp.float32)]
    scratch = (
        pltpu.VMEM((CPT, CHUNK), jnp.int32),
        pltpu.VMEM((CPT, CHUNK), jnp.int32),
        pltpu.VMEM((NBUF * CHUNK, QW), jnp.float32),
        pltpu.VMEM((CHUNK,), jnp.float32),
        pltpu.VMEM((DBUF,), jnp.float32),
        pltpu.VMEM_SHARED((NUP, QW), jnp.float32),
        pltpu.VMEM_SHARED((NUP,), jnp.float32),
        *[pltpu.SemaphoreType.DMA for _ in range(2 * NBUF)],
    )

    def body(tP, sP, dP, tQ, sQ, dQ, zacc, outP, outQ, *rest):
        if with_deg:
            degI, degU = rest[0], rest[1]
            rest = rest[2:]
        else:
            degI = degU = None
        (src_st, dst_st, rows, ones_v, dbuf, acc_sh, deg_sh) = rest[:7]
        sems = rest[7:]
        sg = list(sems[:NBUF])
        ss = list(sems[NBUF:2 * NBUF])
        c = lax.axis_index("c")
        s = lax.axis_index("s")
        if with_deg:
            for j in range(CHUNK // 16):
                ones_v[pl.ds(j * 16, 16)] = jnp.full((16,), 1.0, jnp.float32)
        common = (src_st, dst_st, rows, ones_v, dbuf, acc_sh, deg_sh, sg, ss)
        _agg_pass(c, s, NIP, tP, sP, dP, outP, zacc, degI, 0, *common)
        _agg_pass(c, s, NUP, tQ, sQ, dQ, outQ, zacc, degU, 1, *common)

    return pl.kernel(body, out_type=tuple(out_type), mesh=mesh,
                     scratch_types=scratch,
                     compiler_params=pltpu.CompilerParams(
                         use_tc_tiling_on_sc=False))


BR = 400


def _sage_block(feat_ref, agg_ref, deg_ref, ws_ref, wn_ref, b_ref, o_ref,
                *, normalize):
    r = 1.0 / jnp.maximum(deg_ref[...], 1.0)
    m = agg_ref[...] * r
    y = (jnp.dot(feat_ref[...], ws_ref[...], preferred_element_type=jnp.float32)
         + jnp.dot(m, wn_ref[...], preferred_element_type=jnp.float32)
         + b_ref[...])
    if normalize:
        n = jnp.sqrt(jnp.sum(y * y, axis=1, keepdims=True))
        y = y / jnp.maximum(n, 1e-12)
    o_ref[...] = y


def _sage_tc(feat, agg, deg2, ws, wn, b2, n_rows, off, normalize):
    return pl.pallas_call(
        functools.partial(_sage_block, normalize=normalize),
        grid=(n_rows // BR,),
        in_specs=[
            pl.BlockSpec((BR, D), lambda i, o=off: (i + o, 0)),
            pl.BlockSpec((BR, D), lambda i, o=off: (i + o, 0)),
            pl.BlockSpec((BR, 1), lambda i, o=off: (i + o, 0)),
            pl.BlockSpec((D, D), lambda i: (0, 0)),
            pl.BlockSpec((D, D), lambda i: (0, 0)),
            pl.BlockSpec((1, D), lambda i: (0, 0)),
        ],
        out_specs=pl.BlockSpec((BR, D), lambda i: (i, 0)),
        out_shape=jax.ShapeDtypeStruct((n_rows, D), jnp.float32),
    )(feat, agg, deg2, ws, wn, b2)


def kernel(x_user, x_item, rates_src, rates_dst, rev_src, rev_dst,
           l1_rates_Wself, l1_rates_Wneigh, l1_rates_b,
           l1_rev_Wself, l1_rev_Wneigh, l1_rev_b,
           l2_rates_Wself, l2_rates_Wneigh, l2_rates_b,
           l2_rev_Wself, l2_rev_Wneigh, l2_rev_b):
    ar = jnp.arange(EPAD - E, dtype=jnp.int32)
    rs = jnp.concatenate([rates_src, ar % 64]).reshape(EPAD // CHUNK, CHUNK)
    rd = jnp.concatenate([rates_dst, NI + (ar % (NIP - NI))]).reshape(EPAD // CHUNK, CHUNK)
    vs = jnp.concatenate([rev_src, ar % 64]).reshape(EPAD // CHUNK, CHUNK)
    vd = jnp.concatenate([rev_dst, NU + (ar % (NUP - NU))]).reshape(EPAD // CHUNK, CHUNK)
    zacc = jnp.zeros((NUP, QW), jnp.float32)
    rs, rd, vs, vd, zacc = lax.optimization_barrier((rs, rd, vs, vd, zacc))

    aggP, aggQ, degI, degU = _make_sc_kernel(True)(
        x_user.reshape(NQ * NU, QW), rs, rd,
        x_item.reshape(NQ * NI, QW), vs, vd, zacc)
    degI2 = degI.reshape(NIP, 1)
    degU2 = degU.reshape(NUP, 1)

    item1 = _sage_tc(x_item, aggP.reshape(NIP, D), degI2,
                     l1_rates_Wself, l1_rates_Wneigh,
                     l1_rates_b.reshape(1, D), NI, 0, False)
    user1 = _sage_tc(x_user, aggQ.reshape(NUP, D), degU2,
                     l1_rev_Wself, l1_rev_Wneigh,
                     l1_rev_b.reshape(1, D), NU, 0, False)

    aggS, aggT = _make_sc_kernel(False)(
        user1.reshape(NQ * NU, QW), rs, rd,
        item1.reshape(NQ * NI, QW), vs, vd, zacc)

    item2 = _sage_tc(item1, aggS.reshape(NIP, D), degI2,
                     l2_rates_Wself, l2_rates_Wneigh,
                     l2_rates_b.reshape(1, D), NI, 0, True)
    user2t = _sage_tc(user1, aggT.reshape(NUP, D), degU2,
                      l2_rev_Wself, l2_rev_Wneigh,
                      l2_rev_b.reshape(1, D), NU - NI, NI // BR, True)
    return jnp.concatenate([item2, user2t], axis=0)


# speedup vs baseline: 8.6772x; 1.3181x over previous
import functools

import jax
import jax.numpy as jnp
from jax import lax
from jax.experimental import pallas as pl
from jax.experimental.pallas import tpu as pltpu
from jax.experimental.pallas import tpu_sc as plsc

NU, NI, D, E = 30000, 20000, 128, 300000
NUP, NIP = 30080, 20096
NC, NT = 2, 16
CHUNK = 128
CPT = 152
EPAD = NT * CPT * CHUNK
HALF = D // 2


DBUF = 1888
NQ = 4
QW = D // NQ


NBUF = 6
STAG = 3
SD = 3


def _agg_pass(c, s, n_pad, table4, src2d, dst2d, out_hbm, zacc,
              deg_out, deg_core, src_st, dst_st, rows, ones_v,
              dbuf, acc_sh, deg_sh, sg, ss):
    rows_pt = n_pad // NT
    row_sl = pl.ds(s * rows_pt, rows_pt)
    pltpu.sync_copy(src2d.at[pl.ds(s * CPT, CPT)], src_st)
    pltpu.sync_copy(dst2d.at[pl.ds(s * CPT, CPT)], dst_st)

    for h in range(2):
        q = 2 * h + c
        pltpu.sync_copy(zacc.at[row_sl], acc_sh.at[row_sl])
        if deg_out is not None and h == 0:
            @pl.when(c == deg_core)
            def _():
                def _zb(i, carry):
                    dbuf[pl.ds(i * 16, 16)] = jnp.zeros((16,), jnp.float32)
                    return carry
                lax.fori_loop(0, DBUF // 16, _zb, 0)
                pltpu.sync_copy(dbuf.at[pl.ds(0, rows_pt)], deg_sh.at[row_sl])

        def _trans(i, carry):
            for j in range(CHUNK // 16):
                sl = pl.ds(j * 16, 16)
                v = src_st[i, sl]
                if h == 0:
                    src_st[i, sl] = v * 4 + c
                else:
                    src_st[i, sl] = v + 2
            return carry
        lax.fori_loop(0, CPT, _trans, 0)

        plsc.subcore_barrier()

        def _rbuf(b):
            return rows.at[pl.ds(b * CHUNK, CHUNK)]

        def g_start(i, b):
            pltpu.async_copy(table4.at[src_st.at[i]], _rbuf(b), sg[b])

        def g_wait(i, b):
            pltpu.make_async_copy(table4.at[src_st.at[i]], _rbuf(b),
                                  sg[b]).wait()

        def s_start(i, b):
            pltpu.async_copy(_rbuf(b), acc_sh.at[dst_st.at[i]], ss[b],
                             add=True)
            if deg_out is not None and h == 0:
                @pl.when(c == deg_core)
                def _():
                    pltpu.sync_copy(ones_v, deg_sh.at[dst_st.at[i]],
                                    add=True)

        def s_wait(i, b):
            pltpu.make_async_copy(_rbuf(b), acc_sh.at[dst_st.at[i]],
                                  ss[b]).wait()

        def _step(t, carry):
            for k in range(NBUF):
                i = t * NBUF + k
                @pl.when(jnp.logical_and(i >= STAG + SD,
                                         i < CPT + STAG + SD))
                def _(i=i, k=k):
                    s_wait(i - STAG - SD, k)
                @pl.when(i < CPT)
                def _(i=i, k=k):
                    g_start(i, k)
                b1 = (k - STAG) % NBUF
                @pl.when(jnp.logical_and(i >= STAG, i < CPT + STAG))
                def _(i=i, b1=b1):
                    g_wait(i - STAG, b1)
                    s_start(i - STAG, b1)
            return carry
        lax.fori_loop(0, (CPT + STAG + SD + NBUF - 1) // NBUF, _step, 0)

        plsc.subcore_barrier()
        pltpu.sync_copy(acc_sh.at[row_sl],
                        out_hbm.at[row_sl, pl.ds(q * QW, QW)])
        if deg_out is not None and h == 0:
            @pl.when(c == deg_core)
            def _():
                pltpu.sync_copy(deg_sh.at[row_sl], dbuf.at[pl.ds(0, rows_pt)])
                pltpu.sync_copy(dbuf.at[pl.ds(0, rows_pt)], deg_out.at[row_sl])
        plsc.subcore_barrier()


@functools.lru_cache(maxsize=None)
def _make_sc_kernel(with_deg):
    mesh = plsc.VectorSubcoreMesh(core_axis_name="c", subcore_axis_name="s",
                                  num_cores=NC, num_subcores=NT)
    out_type = [jax.ShapeDtypeStruct((NIP, D), jnp.float32),
                jax.ShapeDtypeStruct((NUP, D), jnp.float32)]
    if with_deg:
        out_type += [jax.ShapeDtypeStruct((NIP,), jnp.float32),
                     jax.ShapeDtypeStruct((NUP,), jnp.float32)]
    scratch = (
        pltpu.VMEM((CPT, CHUNK), jnp.int32),
        pltpu.VMEM((CPT, CHUNK), jnp.int32),
        pltpu.VMEM((NBUF * CHUNK, QW), jnp.float32),
        pltpu.VMEM((CHUNK,), jnp.float32),
        pltpu.VMEM((DBUF,), jnp.float32),
        pltpu.VMEM_SHARED((NUP, QW), jnp.float32),
        pltpu.VMEM_SHARED((NUP,), jnp.float32),
        *[pltpu.SemaphoreType.DMA for _ in range(2 * NBUF)],
    )

    def body(tP, sP, dP, tQ, sQ, dQ, zacc, outP, outQ, *rest):
        if with_deg:
            degI, degU = rest[0], rest[1]
            rest = rest[2:]
        else:
            degI = degU = None
        (src_st, dst_st, rows, ones_v, dbuf, acc_sh, deg_sh) = rest[:7]
        sems = rest[7:]
        sg = list(sems[:NBUF])
        ss = list(sems[NBUF:2 * NBUF])
        c = lax.axis_index("c")
        s = lax.axis_index("s")
        if with_deg:
            for j in range(CHUNK // 16):
                ones_v[pl.ds(j * 16, 16)] = jnp.full((16,), 1.0, jnp.float32)
        common = (src_st, dst_st, rows, ones_v, dbuf, acc_sh, deg_sh, sg, ss)
        _agg_pass(c, s, NIP, tP, sP, dP, outP, zacc, degI, 0, *common)
        _agg_pass(c, s, NUP, tQ, sQ, dQ, outQ, zacc, degU, 1, *common)

    return pl.kernel(body, out_type=tuple(out_type), mesh=mesh,
                     scratch_types=scratch,
                     compiler_params=pltpu.CompilerParams(
                         use_tc_tiling_on_sc=False))


BR = 400


def _sage_block(feat_ref, agg_ref, deg_ref, ws_ref, wn_ref, b_ref, o_ref,
                *, normalize):
    r = 1.0 / jnp.maximum(deg_ref[...], 1.0)
    m = agg_ref[...] * r
    y = (jnp.dot(feat_ref[...], ws_ref[...], preferred_element_type=jnp.float32)
         + jnp.dot(m, wn_ref[...], preferred_element_type=jnp.float32)
         + b_ref[...])
    if normalize:
        n = jnp.sqrt(jnp.sum(y * y, axis=1, keepdims=True))
        y = y / jnp.maximum(n, 1e-12)
    o_ref[...] = y


def _sage_tc(feat, agg, deg2, ws, wn, b2, n_rows, off, normalize):
    return pl.pallas_call(
        functools.partial(_sage_block, normalize=normalize),
        grid=(n_rows // BR,),
        in_specs=[
            pl.BlockSpec((BR, D), lambda i, o=off: (i + o, 0)),
            pl.BlockSpec((BR, D), lambda i, o=off: (i + o, 0)),
            pl.BlockSpec((BR, 1), lambda i, o=off: (i + o, 0)),
            pl.BlockSpec((D, D), lambda i: (0, 0)),
            pl.BlockSpec((D, D), lambda i: (0, 0)),
            pl.BlockSpec((1, D), lambda i: (0, 0)),
        ],
        out_specs=pl.BlockSpec((BR, D), lambda i: (i, 0)),
        out_shape=jax.ShapeDtypeStruct((n_rows, D), jnp.float32),
    )(feat, agg, deg2, ws, wn, b2)


def kernel(x_user, x_item, rates_src, rates_dst, rev_src, rev_dst,
           l1_rates_Wself, l1_rates_Wneigh, l1_rates_b,
           l1_rev_Wself, l1_rev_Wneigh, l1_rev_b,
           l2_rates_Wself, l2_rates_Wneigh, l2_rates_b,
           l2_rev_Wself, l2_rev_Wneigh, l2_rev_b):
    ar = jnp.arange(EPAD - E, dtype=jnp.int32)
    rs = jnp.concatenate([rates_src, ar % 64]).reshape(EPAD // CHUNK, CHUNK)
    rd = jnp.concatenate([rates_dst, NI + (ar % (NIP - NI))]).reshape(EPAD // CHUNK, CHUNK)
    vs = jnp.concatenate([rev_src, ar % 64]).reshape(EPAD // CHUNK, CHUNK)
    vd = jnp.concatenate([rev_dst, NU + (ar % (NUP - NU))]).reshape(EPAD // CHUNK, CHUNK)
    zacc = jnp.zeros((NUP, QW), jnp.float32)
    rs, rd, vs, vd, zacc = lax.optimization_barrier((rs, rd, vs, vd, zacc))

    aggP, aggQ, degI, degU = _make_sc_kernel(True)(
        x_user.reshape(NQ * NU, QW), rs, rd,
        x_item.reshape(NQ * NI, QW), vs, vd, zacc)
    degI2 = degI.reshape(NIP, 1)
    degU2 = degU.reshape(NUP, 1)

    item1 = _sage_tc(x_item, aggP, degI2,
                     l1_rates_Wself, l1_rates_Wneigh,
                     l1_rates_b.reshape(1, D), NI, 0, False)
    user1 = _sage_tc(x_user, aggQ, degU2,
                     l1_rev_Wself, l1_rev_Wneigh,
                     l1_rev_b.reshape(1, D), NU, 0, False)

    aggS, aggT = _make_sc_kernel(False)(
        user1.reshape(NQ * NU, QW), rs, rd,
        item1.reshape(NQ * NI, QW), vs, vd, zacc)

    item2 = _sage_tc(item1, aggS, degI2,
                     l2_rates_Wself, l2_rates_Wneigh,
                     l2_rates_b.reshape(1, D), NI, 0, True)
    user2t = _sage_tc(user1, aggT, degU2,
                      l2_rev_Wself, l2_rev_Wneigh,
                      l2_rev_b.reshape(1, D), NU - NI, NI // BR, True)
    return jnp.concatenate([item2, user2t], axis=0)


# speedup vs baseline: 8.8534x; 1.0203x over previous
import functools

import jax
import jax.numpy as jnp
from jax import lax
from jax.experimental import pallas as pl
from jax.experimental.pallas import tpu as pltpu
from jax.experimental.pallas import tpu_sc as plsc

NU, NI, D, E = 30000, 20000, 128, 300000
NUP, NIP = 30080, 20096
NC, NT = 2, 16
CHUNK = 128
CPT = 152
EPAD = NT * CPT * CHUNK
HALF = D // 2


DBUF = 1888
NQ = 4
QW = D // NQ


NBUF = 6
STAG = 4
SD = 2


def _agg_pass(c, s, n_pad, table4, src2d, dst2d, out_hbm, zacc,
              deg_out, deg_core, src_st, dst_st, rows, ones_v,
              dbuf, acc_sh, deg_sh, sg, ss):
    rows_pt = n_pad // NT
    row_sl = pl.ds(s * rows_pt, rows_pt)
    pltpu.sync_copy(src2d.at[pl.ds(s * CPT, CPT)], src_st)
    pltpu.sync_copy(dst2d.at[pl.ds(s * CPT, CPT)], dst_st)

    for h in range(2):
        q = 2 * h + c
        pltpu.sync_copy(zacc.at[row_sl], acc_sh.at[row_sl])
        if deg_out is not None and h == 0:
            @pl.when(c == deg_core)
            def _():
                def _zb(i, carry):
                    dbuf[pl.ds(i * 16, 16)] = jnp.zeros((16,), jnp.float32)
                    return carry
                lax.fori_loop(0, DBUF // 16, _zb, 0)
                pltpu.sync_copy(dbuf.at[pl.ds(0, rows_pt)], deg_sh.at[row_sl])

        def _trans(i, carry):
            for j in range(CHUNK // 16):
                sl = pl.ds(j * 16, 16)
                v = src_st[i, sl]
                if h == 0:
                    src_st[i, sl] = v * 4 + c
                else:
                    src_st[i, sl] = v + 2
            return carry
        lax.fori_loop(0, CPT, _trans, 0)

        plsc.subcore_barrier()

        def _rbuf(b):
            return rows.at[pl.ds(b * CHUNK, CHUNK)]

        def g_start(i, b):
            pltpu.async_copy(table4.at[src_st.at[i]], _rbuf(b), sg[b])

        def g_wait(i, b):
            pltpu.make_async_copy(table4.at[src_st.at[i]], _rbuf(b),
                                  sg[b]).wait()

        def s_start(i, b):
            pltpu.async_copy(_rbuf(b), acc_sh.at[dst_st.at[i]], ss[b],
                             add=True)
            if deg_out is not None and h == 0:
                @pl.when(c == deg_core)
                def _():
                    pltpu.sync_copy(ones_v, deg_sh.at[dst_st.at[i]],
                                    add=True)

        def s_wait(i, b):
            pltpu.make_async_copy(_rbuf(b), acc_sh.at[dst_st.at[i]],
                                  ss[b]).wait()

        def _step(t, carry):
            for k in range(NBUF):
                i = t * NBUF + k
                @pl.when(jnp.logical_and(i >= STAG + SD,
                                         i < CPT + STAG + SD))
                def _(i=i, k=k):
                    s_wait(i - STAG - SD, k)
                @pl.when(i < CPT)
                def _(i=i, k=k):
                    g_start(i, k)
                b1 = (k - STAG) % NBUF
                @pl.when(jnp.logical_and(i >= STAG, i < CPT + STAG))
                def _(i=i, b1=b1):
                    g_wait(i - STAG, b1)
                    s_start(i - STAG, b1)
            return carry
        lax.fori_loop(0, (CPT + STAG + SD + NBUF - 1) // NBUF, _step, 0)

        plsc.subcore_barrier()
        pltpu.sync_copy(acc_sh.at[row_sl],
                        out_hbm.at[row_sl, pl.ds(q * QW, QW)])
        if deg_out is not None and h == 0:
            @pl.when(c == deg_core)
            def _():
                pltpu.sync_copy(deg_sh.at[row_sl], dbuf.at[pl.ds(0, rows_pt)])
                pltpu.sync_copy(dbuf.at[pl.ds(0, rows_pt)], deg_out.at[row_sl])
        plsc.subcore_barrier()


@functools.lru_cache(maxsize=None)
def _make_sc_kernel(with_deg):
    mesh = plsc.VectorSubcoreMesh(core_axis_name="c", subcore_axis_name="s",
                                  num_cores=NC, num_subcores=NT)
    out_type = [jax.ShapeDtypeStruct((NIP, D), jnp.float32),
                jax.ShapeDtypeStruct((NUP, D), jnp.float32)]
    if with_deg:
        out_type += [jax.ShapeDtypeStruct((NIP,), jnp.float32),
                     jax.ShapeDtypeStruct((NUP,), jnp.float32)]
    scratch = (
        pltpu.VMEM((CPT, CHUNK), jnp.int32),
        pltpu.VMEM((CPT, CHUNK), jnp.int32),
        pltpu.VMEM((NBUF * CHUNK, QW), jnp.float32),
        pltpu.VMEM((CHUNK,), jnp.float32),
        pltpu.VMEM((DBUF,), jnp.float32),
        pltpu.VMEM_SHARED((NUP, QW), jnp.float32),
        pltpu.VMEM_SHARED((NUP,), jnp.float32),
        *[pltpu.SemaphoreType.DMA for _ in range(2 * NBUF)],
    )

    def body(tP, sP, dP, tQ, sQ, dQ, zacc, outP, outQ, *rest):
        if with_deg:
            degI, degU = rest[0], rest[1]
            rest = rest[2:]
        else:
            degI = degU = None
        (src_st, dst_st, rows, ones_v, dbuf, acc_sh, deg_sh) = rest[:7]
        sems = rest[7:]
        sg = list(sems[:NBUF])
        ss = list(sems[NBUF:2 * NBUF])
        c = lax.axis_index("c")
        s = lax.axis_index("s")
        if with_deg:
            for j in range(CHUNK // 16):
                ones_v[pl.ds(j * 16, 16)] = jnp.full((16,), 1.0, jnp.float32)
        common = (src_st, dst_st, rows, ones_v, dbuf, acc_sh, deg_sh, sg, ss)
        _agg_pass(c, s, NIP, tP, sP, dP, outP, zacc, degI, 0, *common)
        _agg_pass(c, s, NUP, tQ, sQ, dQ, outQ, zacc, degU, 1, *common)

    return pl.kernel(body, out_type=tuple(out_type), mesh=mesh,
                     scratch_types=scratch,
                     compiler_params=pltpu.CompilerParams(
                         use_tc_tiling_on_sc=False))


BR = 400


def _sage_block(feat_ref, agg_ref, deg_ref, ws_ref, wn_ref, b_ref, o_ref,
                *, normalize):
    r = 1.0 / jnp.maximum(deg_ref[...], 1.0)
    m = agg_ref[...] * r
    y = (jnp.dot(feat_ref[...], ws_ref[...], preferred_element_type=jnp.float32)
         + jnp.dot(m, wn_ref[...], preferred_element_type=jnp.float32)
         + b_ref[...])
    if normalize:
        n = jnp.sqrt(jnp.sum(y * y, axis=1, keepdims=True))
        y = y / jnp.maximum(n, 1e-12)
    o_ref[...] = y


def _sage_tc(feat, agg, deg2, ws, wn, b2, n_rows, off, normalize):
    return pl.pallas_call(
        functools.partial(_sage_block, normalize=normalize),
        grid=(n_rows // BR,),
        in_specs=[
            pl.BlockSpec((BR, D), lambda i, o=off: (i + o, 0)),
            pl.BlockSpec((BR, D), lambda i, o=off: (i + o, 0)),
            pl.BlockSpec((BR, 1), lambda i, o=off: (i + o, 0)),
            pl.BlockSpec((D, D), lambda i: (0, 0)),
            pl.BlockSpec((D, D), lambda i: (0, 0)),
            pl.BlockSpec((1, D), lambda i: (0, 0)),
        ],
        out_specs=pl.BlockSpec((BR, D), lambda i: (i, 0)),
        out_shape=jax.ShapeDtypeStruct((n_rows, D), jnp.float32),
    )(feat, agg, deg2, ws, wn, b2)


def kernel(x_user, x_item, rates_src, rates_dst, rev_src, rev_dst,
           l1_rates_Wself, l1_rates_Wneigh, l1_rates_b,
           l1_rev_Wself, l1_rev_Wneigh, l1_rev_b,
           l2_rates_Wself, l2_rates_Wneigh, l2_rates_b,
           l2_rev_Wself, l2_rev_Wneigh, l2_rev_b):
    ar = jnp.arange(EPAD - E, dtype=jnp.int32)
    rs = jnp.concatenate([rates_src, ar % 64]).reshape(EPAD // CHUNK, CHUNK)
    rd = jnp.concatenate([rates_dst, NI + (ar % (NIP - NI))]).reshape(EPAD // CHUNK, CHUNK)
    vs = jnp.concatenate([rev_src, ar % 64]).reshape(EPAD // CHUNK, CHUNK)
    vd = jnp.concatenate([rev_dst, NU + (ar % (NUP - NU))]).reshape(EPAD // CHUNK, CHUNK)
    zacc = jnp.zeros((NUP, QW), jnp.float32)
    rs, rd, vs, vd, zacc = lax.optimization_barrier((rs, rd, vs, vd, zacc))

    aggP, aggQ, degI, degU = _make_sc_kernel(True)(
        x_user.reshape(NQ * NU, QW), rs, rd,
        x_item.reshape(NQ * NI, QW), vs, vd, zacc)
    degI2 = degI.reshape(NIP, 1)
    degU2 = degU.reshape(NUP, 1)

    item1 = _sage_tc(x_item, aggP, degI2,
                     l1_rates_Wself, l1_rates_Wneigh,
                     l1_rates_b.reshape(1, D), NI, 0, False)
    user1 = _sage_tc(x_user, aggQ, degU2,
                     l1_rev_Wself, l1_rev_Wneigh,
                     l1_rev_b.reshape(1, D), NU, 0, False)

    aggS, aggT = _make_sc_kernel(False)(
        user1.reshape(NQ * NU, QW), rs, rd,
        item1.reshape(NQ * NI, QW), vs, vd, zacc)

    item2 = _sage_tc(item1, aggS, degI2,
                     l2_rates_Wself, l2_rates_Wneigh,
                     l2_rates_b.reshape(1, D), NI, 0, True)
    user2t = _sage_tc(user1, aggT, degU2,
                      l2_rev_Wself, l2_rev_Wneigh,
                      l2_rev_b.reshape(1, D), NU - NI, NI // BR, True)
    return jnp.concatenate([item2, user2t], axis=0)
